# Initial kernel scaffold; baseline (speedup 1.0000x reference)
import jax
import jax.numpy as jnp
from jax.experimental import pallas as pl


def kernel(x_proposal, x_branch, edge_attr_pp, edge_attr_bp, edge_attr_bb, params, edge_index_pp, edge_index_bp, edge_index_bb):
    raise NotImplementedError("write your pallas kernel here")


# speedup vs baseline: 13.8408x; 13.8408x over previous
import functools

import jax
import jax.numpy as jnp
from jax import lax
from jax.experimental import pallas as pl
from jax.experimental.pallas import tpu as pltpu
from jax.experimental.pallas import tpu_sc as plsc

N = 10000
NPAD = 10240
E = 320000
H = 128
NC, NS = 2, 16
NW = NC * NS
EPW = E // NW
BCH = 2000
SUB = 80
KC = 80
SPW = 320
SSL = 320
STRIPE = NPAD // NS

_f32 = jnp.float32
_i32 = jnp.int32


def _lr(x, s):
    return jnp.where(x >= 0, x, x * s)


def _prep1_body(xp, xb, winp, binp, winb, binb,
                wpp, aspp, adpp, wsbp, asbp, wdbp, adbp, wbb, asbb, adbb,
                hpp_o, hbp_o, hbb_o, s0_o, s1_o, s2_o, s3_o, s4_o, s5_o,
                maxs_o):
    i = pl.program_id(0)
    hp = _lr(xp[...] @ winp[...] + binp[...], 0.01)
    hb = _lr(xb[...] @ winb[...] + binb[...], 0.01)
    hpp = hp @ wpp[...]
    hbp = hb @ wsbp[...]
    hbb = hb @ wbb[...]
    hpp_o[...] = hpp
    hbp_o[...] = hbp
    hbb_o[...] = hbb
    s0 = hpp @ aspp[...]
    s1 = hpp @ adpp[...]
    s2 = hbp @ asbp[...]
    s3 = hp @ (wdbp[...] @ adbp[...])
    s4 = hbb @ asbb[...]
    s5 = hbb @ adbb[...]
    s0_o[...] = s0
    s1_o[...] = s1
    s2_o[...] = s2
    s3_o[...] = s3
    s4_o[...] = s4
    s5_o[...] = s5
    @pl.when(i == 0)
    def _():
        maxs_o[...] = jnp.full((8, H), -1e30, _f32)

    rows = [jnp.full((1, H), jnp.max(s), _f32) for s in (s0, s1, s2, s3, s4, s5)]
    rows.append(jnp.full((2, H), -1e30, _f32))
    maxs_o[...] = jnp.maximum(maxs_o[...], jnp.concatenate(rows, axis=0))


def _prep1(xp, xb, winp, binp, winb, binb,
           wpp, aspp, adpp, wsbp, asbp, wdbp, adbp, wbb, asbb, adbb):
    nblk = NPAD // _MR
    full0 = lambda s: pl.BlockSpec(s, lambda i: (0,) * len(s))
    row = pl.BlockSpec((_MR, H), lambda i: (i, 0))
    col = pl.BlockSpec((_MR, 1), lambda i: (i, 0))
    consts = [winp, binp, winb, binb,
              wpp, aspp, adpp, wsbp, asbp, wdbp, adbp, wbb, asbb, adbb]
    out_shape = [jax.ShapeDtypeStruct((NPAD, H), _f32)] * 3 + \
                [jax.ShapeDtypeStruct((NPAD, 1), _f32)] * 6 + \
                [jax.ShapeDtypeStruct((8, H), _f32)]
    return pl.pallas_call(
        _prep1_body,
        grid=(nblk,),
        in_specs=[pl.BlockSpec((_MR, xp.shape[1]), lambda i: (i, 0)),
                  pl.BlockSpec((_MR, xb.shape[1]), lambda i: (i, 0))] +
                 [full0(c.shape) for c in consts],
        out_specs=[row, row, row] + [col] * 6 +
                  [pl.BlockSpec((8, H), lambda i: (0, 0))],
        out_shape=out_shape,
    )(xp, xb, *consts)


_ET = 512


def _edge_body(ea_t, wet, be, we1, at1, we2, at2, a1_o, a2_o, m1_o, m2_o):
    i = pl.program_id(0)
    v1 = we1[...] @ at1[...]
    v2 = we2[...] @ at2[...]
    z = _lr(wet[...] @ ea_t[...] + be[...], 0.01)
    a1 = jnp.sum(z * v1, axis=0)
    a2 = jnp.sum(z * v2, axis=0)
    a1_o[...] = a1
    a2_o[...] = a2

    @pl.when(i == 0)
    def _():
        m1_o[...] = jnp.full((1, H), -1e30, _f32)
        m2_o[...] = jnp.full((1, H), -1e30, _f32)

    m1_o[...] = jnp.maximum(m1_o[...], jnp.max(a1))
    m2_o[...] = jnp.maximum(m2_o[...], jnp.max(a2))


def _edge_prep(ea_t, wet, be, we1, at1, we2, at2):
    nblk = E // _ET
    full0 = lambda s: pl.BlockSpec(s, lambda i: (0,) * len(s))
    out_shape = [jax.ShapeDtypeStruct((E,), _f32),
                 jax.ShapeDtypeStruct((E,), _f32),
                 jax.ShapeDtypeStruct((1, H), _f32),
                 jax.ShapeDtypeStruct((1, H), _f32)]
    return pl.pallas_call(
        _edge_body,
        grid=(nblk,),
        in_specs=[pl.BlockSpec((16, _ET), lambda i: (0, i)),
                  full0(wet.shape), full0(be.shape), full0(we1.shape),
                  full0(at1.shape), full0(we2.shape), full0(at2.shape)],
        out_specs=[pl.BlockSpec((_ET,), lambda i: (i,)),
                   pl.BlockSpec((_ET,), lambda i: (i,)),
                   pl.BlockSpec((1, H), lambda i: (0, 0)),
                   pl.BlockSpec((1, H), lambda i: (0, 0))],
        out_shape=out_shape,
    )(ea_t, wet, be, we1, at1, we2, at2)


_MR = 1280


def _mid_body(ppo, bpo, bbo, b0, b1, b2,
              wpp, aspp, adpp, wsbp, asbp, wdbp, adbp, wbb, asbb, adbb,
              hpp_o, hbp_o, hbb_o, s0_o, s1_o, s2_o, s3_o, s4_o, s5_o,
              maxs_o):
    i = pl.program_id(0)
    op = ppo[0] + ppo[1] + bpo[0] + bpo[1] + b0[...] + b1[...]
    ob = bbo[0] + bbo[1] + b2[...]
    hpp = op @ wpp[...]
    hbp = ob @ wsbp[...]
    hbb = ob @ wbb[...]
    hpp_o[...] = hpp
    hbp_o[...] = hbp
    hbb_o[...] = hbb
    s0 = hpp @ aspp[...]
    s1 = hpp @ adpp[...]
    s2 = hbp @ asbp[...]
    s3 = op @ (wdbp[...] @ adbp[...])
    s4 = hbb @ asbb[...]
    s5 = hbb @ adbb[...]
    for ref, val in ((s0_o, s0), (s1_o, s1), (s2_o, s2),
                     (s3_o, s3), (s4_o, s4), (s5_o, s5)):
        ref[...] = val

    @pl.when(i == 0)
    def _():
        maxs_o[...] = jnp.full((8, H), -1e30, _f32)

    rows = [jnp.full((1, H), jnp.max(s), _f32) for s in (s0, s1, s2, s3, s4, s5)]
    rows.append(jnp.full((2, H), -1e30, _f32))
    maxs_o[...] = jnp.maximum(maxs_o[...], jnp.concatenate(rows, axis=0))


def _mid(ppo, bpo, bbo, b0, b1, b2,
         wpp, aspp, adpp, wsbp, asbp, wdbp, adbp, wbb, asbb, adbb):
    nblk = NPAD // _MR
    parts = pl.BlockSpec((2, _MR, H), lambda i: (0, i, 0))
    full0 = lambda s: pl.BlockSpec(s, lambda i: (0,) * len(s))
    row = pl.BlockSpec((_MR, H), lambda i: (i, 0))
    col = pl.BlockSpec((_MR, 1), lambda i: (i, 0))
    consts = [b0, b1, b2, wpp, aspp, adpp, wsbp, asbp, wdbp, adbp,
              wbb, asbb, adbb]
    out_shape = [jax.ShapeDtypeStruct((NPAD, H), _f32)] * 3 + \
                [jax.ShapeDtypeStruct((NPAD, 1), _f32)] * 6 + \
                [jax.ShapeDtypeStruct((8, H), _f32)]
    return pl.pallas_call(
        _mid_body,
        grid=(nblk,),
        in_specs=[parts, parts, parts] + [full0(c.shape) for c in consts],
        out_specs=[row, row, row] + [col] * 6 +
                  [pl.BlockSpec((8, H), lambda i: (0, 0))],
        out_shape=out_shape,
    )(ppo, bpo, bbo, *consts)


def _final_body(ppo, bpo, b0, b1, wout, bout, out_o):
    op2 = ppo[0] + ppo[1] + bpo[0] + bpo[1] + b0[...] + b1[...]
    out_o[...] = op2 @ wout[...] + bout[...]


def _final(ppo, bpo, b0, b1, wout, bout):
    nblk = NPAD // _MR
    parts = pl.BlockSpec((2, _MR, H), lambda i: (0, i, 0))
    full0 = lambda s: pl.BlockSpec(s, lambda i: (0,) * len(s))
    return pl.pallas_call(
        _final_body,
        grid=(nblk,),
        in_specs=[parts, parts, full0(b0.shape), full0(b1.shape),
                  full0(wout.shape), full0(bout.shape)],
        out_specs=pl.BlockSpec((_MR, 1), lambda i: (i, 0)),
        out_shape=jax.ShapeDtypeStruct((NPAD, 1), _f32),
    )(ppo, bpo, b0, b1, wout, bout)


_MESH = plsc.VectorSubcoreMesh(core_axis_name="c", subcore_axis_name="s",
                               num_cores=NC, num_subcores=NS)
_SC_PARAMS = pltpu.CompilerParams(needs_layout_passes=False)


def _zero16():
    return jnp.zeros((16,), _f32)


def _passA_body(dst_h, a1_h, a2_h, degp, s1p, s2p,
                dstbuf, a1buf, a2buf, onesb, sidx, sval, stripe,
                deg_sh, s1_sh, s2_sh):
    cid = lax.axis_index("c")
    sid = lax.axis_index("s")
    wid = cid * NS + sid

    def zloop(i, _):
        stripe[pl.ds(i * 16, 16)] = _zero16()
        return 0
    lax.fori_loop(0, STRIPE // 16, zloop, 0)
    pltpu.sync_copy(stripe, deg_sh.at[pl.ds(sid * STRIPE, STRIPE)])
    pltpu.sync_copy(stripe, s1_sh.at[pl.ds(sid * STRIPE, STRIPE)])
    pltpu.sync_copy(stripe, s2_sh.at[pl.ds(sid * STRIPE, STRIPE)])

    def oloop(i, _):
        onesb[pl.ds(i * 16, 16)] = jnp.ones((16,), _f32)
        return 0
    lax.fori_loop(0, SUB // 16, oloop, 0)
    plsc.subcore_barrier()

    def chunk(k, _):
        base = wid * EPW + k * BCH
        pltpu.sync_copy(dst_h.at[pl.ds(base, BCH)], dstbuf)
        pltpu.sync_copy(a1_h.at[pl.ds(base, BCH)], a1buf)
        pltpu.sync_copy(a2_h.at[pl.ds(base, BCH)], a2buf)

        def sub(i, _):
            def lane(j, _):
                off = i * SUB + j * 16
                sidx[pl.ds(j * 16, 16)] = dstbuf[pl.ds(off, 16)]
                return 0
            lax.fori_loop(0, SUB // 16, lane, 0)
            pltpu.sync_copy(onesb, deg_sh.at[sidx], add=True)

            def lane1(j, _):
                off = i * SUB + j * 16
                sval[pl.ds(j * 16, 16)] = a1buf[pl.ds(off, 16)]
                return 0
            lax.fori_loop(0, SUB // 16, lane1, 0)
            pltpu.sync_copy(sval, s1_sh.at[sidx], add=True)

            def lane2(j, _):
                off = i * SUB + j * 16
                sval[pl.ds(j * 16, 16)] = a2buf[pl.ds(off, 16)]
                return 0
            lax.fori_loop(0, SUB // 16, lane2, 0)
            pltpu.sync_copy(sval, s2_sh.at[sidx], add=True)
            return 0
        lax.fori_loop(0, BCH // SUB, sub, 0)
        return 0
    lax.fori_loop(0, EPW // BCH, chunk, 0)
    plsc.subcore_barrier()
    sl = pl.ds(sid * STRIPE, STRIPE)
    pltpu.sync_copy(deg_sh.at[sl], degp.at[cid, sl])
    pltpu.sync_copy(s1_sh.at[sl], s1p.at[cid, sl])
    pltpu.sync_copy(s2_sh.at[sl], s2p.at[cid, sl])


_passA = functools.partial(
    pl.kernel,
    out_type=[jax.ShapeDtypeStruct((NC, NPAD), _f32)] * 3,
    mesh=_MESH,
    compiler_params=_SC_PARAMS,
    scratch_types=[pltpu.VMEM((BCH,), _i32), pltpu.VMEM((BCH,), _f32),
                   pltpu.VMEM((BCH,), _f32), pltpu.VMEM((SUB,), _f32),
                   pltpu.VMEM((SUB,), _i32), pltpu.VMEM((SUB,), _f32),
                   pltpu.VMEM((STRIPE,), _f32),
                   pltpu.VMEM_SHARED((NPAD,), _f32),
                   pltpu.VMEM_SHARED((NPAD,), _f32),
                   pltpu.VMEM_SHARED((NPAD,), _f32)],
)(_passA_body)


def _passB_body(same, *refs):
    if same:
        (src_h, dst_h, ae_h, asrc_h, adst_h, consts_h, degp_h, sp_h,
         denomp, ex_h, exs_h,
         asrc_v, adst_v, m_v, pa, pb, aself_v,
         srcbuf, dstbuf, aebuf, exbuf, sidx, sval, selfex, constsv, stripe,
         denom_sh) = refs
    else:
        (src_h, dst_h, ae_h, asrc_h, adst_h, consts_h,
         denomp, ex_h,
         asrc_v, adst_v, m_v, pa,
         srcbuf, dstbuf, aebuf, exbuf, sidx, sval, constsv, stripe,
         denom_sh) = refs
    cid = lax.axis_index("c")
    sid = lax.axis_index("s")
    wid = cid * NS + sid

    pltpu.sync_copy(consts_h, constsv)
    cmax = constsv[pl.ds(0, 16)][0]
    pltpu.sync_copy(asrc_h, asrc_v)
    pltpu.sync_copy(adst_h, adst_v)

    def mloop(i, _):
        sl = pl.ds(i * 16, 16)
        sv = adst_v[sl] + cmax
        m_v[sl] = _lr(sv, 0.2)
        return 0
    lax.fori_loop(0, NPAD // 16, mloop, 0)

    if same:
        pltpu.sync_copy(degp_h.at[0], pa)
        pltpu.sync_copy(degp_h.at[1], pb)

        def dloop(i, _):
            sl = pl.ds(i * 16, 16)
            pa[sl] = jnp.maximum(pa[sl] + pb[sl], 1.0)
            return 0
        lax.fori_loop(0, NPAD // 16, dloop, 0)
        pltpu.sync_copy(sp_h.at[0], pb)
        pltpu.sync_copy(sp_h.at[1], aself_v)

        def aloop(i, _):
            sl = pl.ds(i * 16, 16)
            aself_v[sl] = (pb[sl] + aself_v[sl]) / pa[sl]
            return 0
        lax.fori_loop(0, NPAD // 16, aloop, 0)

    def zloop(i, _):
        stripe[pl.ds(i * 16, 16)] = _zero16()
        return 0
    lax.fori_loop(0, STRIPE // 16, zloop, 0)
    pltpu.sync_copy(stripe, denom_sh.at[pl.ds(sid * STRIPE, STRIPE)])
    plsc.subcore_barrier()

    def chunk(k, _):
        base = wid * EPW + k * BCH
        pltpu.sync_copy(src_h.at[pl.ds(base, BCH)], srcbuf)
        pltpu.sync_copy(dst_h.at[pl.ds(base, BCH)], dstbuf)
        pltpu.sync_copy(ae_h.at[pl.ds(base, BCH)], aebuf)

        def sub(i, _):
            def lane(j, _):
                off = i * SUB + j * 16
                si = srcbuf[pl.ds(off, 16)]
                di = dstbuf[pl.ds(off, 16)]
                av = plsc.load_gather(asrc_v, [si])
                dv = plsc.load_gather(adst_v, [di])
                mm = plsc.load_gather(m_v, [di])
                sv = av + dv + aebuf[pl.ds(off, 16)]
                ex = jnp.exp(_lr(sv, 0.2) - mm)
                exbuf[pl.ds(off, 16)] = ex
                sidx[pl.ds(j * 16, 16)] = di
                sval[pl.ds(j * 16, 16)] = ex
                return 0
            lax.fori_loop(0, SUB // 16, lane, 0)
            pltpu.sync_copy(sval, denom_sh.at[sidx], add=True)
            return 0
        lax.fori_loop(0, BCH // SUB, sub, 0)
        pltpu.sync_copy(exbuf, ex_h.at[pl.ds(base, BCH)])
        return 0
    lax.fori_loop(0, EPW // BCH, chunk, 0)

    if same:
        iota = lax.iota(_i32, 16)

        def sgroup(g, _):
            def lane(j, _):
                k = g * SUB + j * 16
                n = wid * SPW + k + iota
                valid = n < N
                nc = jnp.where(valid, n, 0)
                av = plsc.load_gather(asrc_v, [nc])
                dv = plsc.load_gather(adst_v, [nc])
                af = plsc.load_gather(aself_v, [nc])
                mm = plsc.load_gather(m_v, [nc])
                sv = av + dv + af
                ex = jnp.exp(_lr(sv, 0.2) - mm)
                ex = jnp.where(valid, ex, 0.0)
                selfex[pl.ds(k, 16)] = ex
                sidx[pl.ds(j * 16, 16)] = nc
                sval[pl.ds(j * 16, 16)] = ex
                return 0
            lax.fori_loop(0, SUB // 16, lane, 0)
            pltpu.sync_copy(sval, denom_sh.at[sidx], add=True)
            return 0
        lax.fori_loop(0, SSL // SUB, sgroup, 0)
        pltpu.sync_copy(selfex, exs_h.at[wid])

    plsc.subcore_barrier()
    sl = pl.ds(sid * STRIPE, STRIPE)
    pltpu.sync_copy(denom_sh.at[sl], denomp.at[cid, sl])


_passB_same = functools.partial(
    pl.kernel,
    out_type=[jax.ShapeDtypeStruct((NC, NPAD), _f32),
              jax.ShapeDtypeStruct((E,), _f32),
              jax.ShapeDtypeStruct((NW, SSL), _f32)],
    mesh=_MESH,
    compiler_params=_SC_PARAMS,
    scratch_types=[pltpu.VMEM((NPAD,), _f32)] * 3 +
                  [pltpu.VMEM((NPAD,), _f32)] * 3 +
                  [pltpu.VMEM((BCH,), _i32), pltpu.VMEM((BCH,), _i32),
                   pltpu.VMEM((BCH,), _f32), pltpu.VMEM((BCH,), _f32),
                   pltpu.VMEM((SUB,), _i32), pltpu.VMEM((SUB,), _f32),
                   pltpu.VMEM((SSL,), _f32), pltpu.VMEM((16,), _f32),
                   pltpu.VMEM((STRIPE,), _f32),
                   pltpu.VMEM_SHARED((NPAD,), _f32)],
)(functools.partial(_passB_body, True))

_passB_bip = functools.partial(
    pl.kernel,
    out_type=[jax.ShapeDtypeStruct((NC, NPAD), _f32),
              jax.ShapeDtypeStruct((E,), _f32)],
    mesh=_MESH,
    compiler_params=_SC_PARAMS,
    scratch_types=[pltpu.VMEM((NPAD,), _f32)] * 4 +
                  [pltpu.VMEM((BCH,), _i32), pltpu.VMEM((BCH,), _i32),
                   pltpu.VMEM((BCH,), _f32), pltpu.VMEM((BCH,), _f32),
                   pltpu.VMEM((SUB,), _i32), pltpu.VMEM((SUB,), _f32),
                   pltpu.VMEM((16,), _f32),
                   pltpu.VMEM((STRIPE,), _f32),
                   pltpu.VMEM_SHARED((NPAD,), _f32)],
)(functools.partial(_passB_body, False))


def _passC_body(same, *refs):
    if same:
        (src_h, dst_h, ex_h, denomp_h, h_h, exs_h, outp,
         rcp_v, pa, srcbuf, dstbuf, exbuf, wbuf, rows, selfex, sem,
         acc_sh) = refs
    else:
        (src_h, dst_h, ex_h, denomp_h, h_h, outp,
         rcp_v, pa, srcbuf, dstbuf, exbuf, wbuf, rows, sem,
         acc_sh) = refs
    cid = lax.axis_index("c")
    sid = lax.axis_index("s")
    wid = cid * NS + sid

    pltpu.sync_copy(denomp_h.at[0], rcp_v)
    pltpu.sync_copy(denomp_h.at[1], pa)

    def rloop(i, _):
        sl = pl.ds(i * 16, 16)
        rcp_v[sl] = 1.0 / (rcp_v[sl] + pa[sl] + 1e-16)
        return 0
    lax.fori_loop(0, NPAD // 16, rloop, 0)

    def zrow(i, _):
        for q in range(H // 16):
            rows[i, pl.ds(q * 16, 16)] = _zero16()
        return 0
    lax.fori_loop(0, KC, zrow, 0)
    for t in range(STRIPE // KC):
        pltpu.sync_copy(rows, acc_sh.at[pl.ds(sid * STRIPE + t * KC, KC)])
    plsc.subcore_barrier()

    def scale_rows(nrows):
        def mrow(j, _):
            wv = wbuf[pl.ds(j * 16, 16)]
            for l in range(16):
                i = j * 16 + l
                ws = wv[l]
                for q in range(H // 16):
                    sl = pl.ds(q * 16, 16)
                    rows[i, sl] = rows[i, sl] * ws
            return 0
        lax.fori_loop(0, nrows // 16, mrow, 0)

    def chunk(k, _):
        base = wid * EPW + k * KC
        pltpu.sync_copy(src_h.at[pl.ds(base, KC)], srcbuf)
        pltpu.sync_copy(dst_h.at[pl.ds(base, KC)], dstbuf)
        pltpu.sync_copy(ex_h.at[pl.ds(base, KC)], exbuf)
        pltpu.async_copy(h_h.at[srcbuf], rows, sem).wait()

        def wl(j, _):
            sl = pl.ds(j * 16, 16)
            di = dstbuf[sl]
            rv = plsc.load_gather(rcp_v, [di])
            wbuf[sl] = exbuf[sl] * rv
            return 0
        lax.fori_loop(0, KC // 16, wl, 0)
        scale_rows(KC)
        pltpu.sync_copy(rows, acc_sh.at[dstbuf], add=True)
        return 0
    lax.fori_loop(0, EPW // KC, chunk, 0)

    if same:
        pltpu.sync_copy(exs_h.at[wid], selfex)
        iota = lax.iota(_i32, 16)

        def sgroup(g, _):
            base_n = wid * SPW + g * SUB
            pltpu.sync_copy(h_h.at[pl.ds(base_n, SUB)], rows)

            def lane(j, _):
                k = g * SUB + j * 16
                n = wid * SPW + k + iota
                valid = n < N
                nc = jnp.where(valid, n, 0)
                rv = plsc.load_gather(rcp_v, [nc])
                wbuf[pl.ds(j * 16, 16)] = selfex[pl.ds(k, 16)] * rv
                dstbuf[pl.ds(j * 16, 16)] = nc
                return 0
            lax.fori_loop(0, SUB // 16, lane, 0)
            scale_rows(SUB)
            pltpu.sync_copy(rows, acc_sh.at[dstbuf], add=True)
            return 0
        lax.fori_loop(0, SSL // SUB, sgroup, 0)

    plsc.subcore_barrier()
    sl = pl.ds(sid * STRIPE, STRIPE)
    pltpu.sync_copy(acc_sh.at[sl], outp.at[cid, sl])


_passC_same = functools.partial(
    pl.kernel,
    out_type=[jax.ShapeDtypeStruct((NC, NPAD, H), _f32)],
    mesh=_MESH,
    compiler_params=_SC_PARAMS,
    scratch_types=[pltpu.VMEM((NPAD,), _f32), pltpu.VMEM((NPAD,), _f32),
                   pltpu.VMEM((KC,), _i32), pltpu.VMEM((KC,), _i32),
                   pltpu.VMEM((KC,), _f32), pltpu.VMEM((KC,), _f32),
                   pltpu.VMEM((KC, H), _f32), pltpu.VMEM((SSL,), _f32),
                   pltpu.SemaphoreType.DMA,
                   pltpu.VMEM_SHARED((NPAD, H), _f32)],
)(functools.partial(_passC_body, True))

_passC_bip = functools.partial(
    pl.kernel,
    out_type=[jax.ShapeDtypeStruct((NC, NPAD, H), _f32)],
    mesh=_MESH,
    compiler_params=_SC_PARAMS,
    scratch_types=[pltpu.VMEM((NPAD,), _f32), pltpu.VMEM((NPAD,), _f32),
                   pltpu.VMEM((KC,), _i32), pltpu.VMEM((KC,), _i32),
                   pltpu.VMEM((KC,), _f32), pltpu.VMEM((KC,), _f32),
                   pltpu.VMEM((KC, H), _f32),
                   pltpu.SemaphoreType.DMA,
                   pltpu.VMEM_SHARED((NPAD, H), _f32)],
)(functools.partial(_passC_body, False))


def kernel(x_proposal, x_branch, edge_attr_pp, edge_attr_bp, edge_attr_bb,
           params, edge_index_pp, edge_index_bp, edge_index_bb):
    p = params
    pad_n = ((0, NPAD - N), (0, 0))
    xp = jnp.pad(x_proposal.astype(_f32), pad_n)
    xb = jnp.pad(x_branch.astype(_f32), pad_n)
    src_pp = edge_index_pp[0].astype(_i32)
    dst_pp = edge_index_pp[1].astype(_i32)
    src_bp = edge_index_bp[0].astype(_i32)
    dst_bp = edge_index_bp[1].astype(_i32)
    src_bb = edge_index_bb[0].astype(_i32)
    dst_bb = edge_index_bb[1].astype(_i32)

    col = lambda v: v.reshape(H, 1)
    row = lambda v: v.reshape(1, -1)

    def gat_w(g):
        return g['W'], col(g['att_src']), col(g['att_dst'])

    g1pp, g1bp, g1bb = p['gat1_pp'], p['gat1_bp'], p['gat1_bb']
    g2pp, g2bp, g2bb = p['gat2_pp'], p['gat2_bp'], p['gat2_bb']

    wpp, aspp, adpp = gat_w(g1pp)
    wbb, asbb, adbb = gat_w(g1bb)
    (hpp1, hbp1, hbb1, s0, s1, s2, s3, s4, s5, maxs1) = _prep1(
        xp, xb, p['W_in_p'], row(p['b_in_p']), p['W_in_b'], row(p['b_in_b']),
        wpp, aspp, adpp, g1bp['W_src'], col(g1bp['att_src']),
        g1bp['W_dst'], col(g1bp['att_dst']), wbb, asbb, adbb)
    scal1 = [v.reshape(NPAD) for v in (s0, s1, s2, s3, s4, s5)]

    def edges(ea, we, be, gl1, gl2):
        return _edge_prep(ea.T.astype(_f32), we.T, col(be).astype(_f32),
                          gl1['W_edge'], col(gl1['att_edge']),
                          gl2['W_edge'], col(gl2['att_edge']))

    ae_pp1, ae_pp2, mpp1, mpp2 = edges(edge_attr_pp, p['W_e_pp'], p['b_e_pp'], g1pp, g2pp)
    ae_bp1, ae_bp2, mbp1, mbp2 = edges(edge_attr_bp, p['W_e_bp'], p['b_e_bp'], g1bp, g2bp)
    ae_bb1, ae_bb2, mbb1, mbb2 = edges(edge_attr_bb, p['W_e_bb'], p['b_e_bb'], g1bb, g2bb)

    deg_pp, spp1, spp2 = _passA(dst_pp, ae_pp1, ae_pp2)
    deg_bb, sbb1, sbb2 = _passA(dst_bb, ae_bb1, ae_bb2)

    c16 = lambda mn, me: jnp.full((16,), mn + me[0, 0], _f32)

    def layer(scal, maxs, h_pp, h_bp, ae_pp, ae_bp, m_pp, m_bp, s_pp,
              h_bb=None, ae_bb=None, m_bb=None, s_bb=None):
        cpp = c16(maxs[0, 0], m_pp)
        cbp = c16(maxs[2, 0], m_bp)
        den_pp, ex_pp, exs_pp = _passB_same(
            src_pp, dst_pp, ae_pp, scal[0], scal[1], cpp, deg_pp, s_pp)
        den_bp, ex_bp = _passB_bip(
            src_bp, dst_bp, ae_bp, scal[2], scal[3], cbp)
        (out_pp,) = _passC_same(src_pp, dst_pp, ex_pp, den_pp, h_pp, exs_pp)
        (out_bp,) = _passC_bip(src_bp, dst_bp, ex_bp, den_bp, h_bp)
        out_bb = None
        if h_bb is not None:
            cbb = c16(maxs[4, 0], m_bb)
            den_bb, ex_bb, exs_bb = _passB_same(
                src_bb, dst_bb, ae_bb, scal[4], scal[5], cbb, deg_bb, s_bb)
            (out_bb,) = _passC_same(src_bb, dst_bb, ex_bb, den_bb, h_bb, exs_bb)
        return out_pp, out_bp, out_bb

    out_pp1, out_bp1, out_bb1 = layer(
        scal1, maxs1, hpp1, hbp1, ae_pp1, ae_bp1, mpp1, mbp1, spp1,
        h_bb=hbb1, ae_bb=ae_bb1, m_bb=mbb1, s_bb=sbb1)

    wpp2, aspp2, adpp2 = gat_w(g2pp)
    wbb2, asbb2, adbb2 = gat_w(g2bb)
    (hpp2, hbp2, hbb2, t0, t1, t2, t3, t4, t5, maxs2) = _mid(
        out_pp1, out_bp1, out_bb1,
        row(g1pp['bias']), row(g1bp['bias']), row(g1bb['bias']),
        wpp2, aspp2, adpp2, g2bp['W_src'], col(g2bp['att_src']),
        g2bp['W_dst'], col(g2bp['att_dst']), wbb2, asbb2, adbb2)
    scal2 = [v.reshape(NPAD) for v in (t0, t1, t2, t3, t4, t5)]

    out_pp2, out_bp2, _ = layer(
        scal2, maxs2, hpp2, hbp2, ae_pp2, ae_bp2, mpp2, mbp2, spp2)

    out = _final(out_pp2, out_bp2, row(g2pp['bias']), row(g2bp['bias']),
                 p['W_out'], row(p['b_out']))
    return out[:N]
